# Initial kernel scaffold; baseline (speedup 1.0000x reference)
import jax
import jax.numpy as jnp
from jax.experimental import pallas as pl


def kernel(vecs, loss_mask, w, c_count):
    raise NotImplementedError("write your pallas kernel here")


# speedup vs baseline: 1.6662x; 1.6662x over previous
import functools

import jax
import jax.numpy as jnp
from jax import lax
from jax.experimental import pallas as pl
from jax.experimental.pallas import tpu as pltpu

B, H, S, D = 4, 8, 1024, 64
L = 1024
C_GAMMA = 0.99
EPS = 0.01
TS = 512
T = S // TS


def _a_body(vecs_ref, mask_ref, w_ref, cnt_ref,
            z_ref, errs_ref, cb_ref, gidx_ref, counts_ref):
    h = pl.program_id(0)
    b = pl.program_id(1)
    t = pl.program_id(2)

    @pl.when((b == 0) & (t == 0))
    def _init():
        cb_ref[0] = w_ref[0] / (cnt_ref[0, 0][:, None] + EPS)
        counts_ref[0, 0] = jnp.zeros((L,), jnp.float32)

    cb = cb_ref[0]
    cbn = jnp.sum(cb * cb, axis=1)
    v = vecs_ref[0, 0]
    vn = jnp.sum(v * v, axis=1, keepdims=True)
    scores = lax.dot_general(v, cb, (((1,), (1,)), ((), ())),
                             preferred_element_type=jnp.float32)
    d2 = (vn - 2.0 * scores) + cbn[None, :]
    m = jnp.min(d2, axis=1, keepdims=True)
    iota = lax.broadcasted_iota(jnp.int32, (TS, L), 1)
    z = jnp.min(jnp.where(d2 == m, iota, L), axis=1)
    z_ref[0, 0, 0, 0] = z
    gidx_ref[0, 0, 0, 0] = z + h * L
    errs_ref[0, 0, 0, 0] = m[:, 0]
    msk = mask_ref[0, 0, 0]
    onehot = jnp.where(z[:, None] == iota, msk[:, None], 0.0)
    counts_ref[0, 0] += jnp.sum(onehot, axis=0)


def _run_a(vecs, mask4, w, c_count):
    grid = (H, B, T)
    out_shapes = (
        jax.ShapeDtypeStruct((B, H, T, 1, TS), jnp.int32),
        jax.ShapeDtypeStruct((B, H, T, 1, TS), jnp.float32),
        jax.ShapeDtypeStruct((H, L, D), jnp.float32),
        jax.ShapeDtypeStruct((B, H, T, 1, TS), jnp.int32),
        jax.ShapeDtypeStruct((H, 1, L), jnp.float32),
    )
    in_specs = [
        pl.BlockSpec((1, 1, TS, D), lambda h, b, t: (b, h, t, 0)),
        pl.BlockSpec((1, 1, 1, TS), lambda h, b, t: (b, t, 0, 0)),
        pl.BlockSpec((1, L, D), lambda h, b, t: (h, 0, 0)),
        pl.BlockSpec((1, 1, L), lambda h, b, t: (h, 0, 0)),
    ]
    out_specs = (
        pl.BlockSpec((1, 1, 1, 1, TS), lambda h, b, t: (b, h, t, 0, 0)),
        pl.BlockSpec((1, 1, 1, 1, TS), lambda h, b, t: (b, h, t, 0, 0)),
        pl.BlockSpec((1, L, D), lambda h, b, t: (h, 0, 0)),
        pl.BlockSpec((1, 1, 1, 1, TS), lambda h, b, t: (b, h, t, 0, 0)),
        pl.BlockSpec((1, 1, L), lambda h, b, t: (h, 0, 0)),
    )
    return pl.pallas_call(
        _a_body,
        grid=grid,
        in_specs=in_specs,
        out_specs=out_specs,
        out_shape=out_shapes,
    )(vecs, mask4, w, c_count.reshape(H, 1, L))


def kernel(vecs, loss_mask, w, c_count):
    mask4 = loss_mask.reshape(B, T, 1, TS)
    z5, errs5, cb, gidx5, counts3 = _run_a(vecs, mask4, w, c_count)
    z = z5.reshape(B, H, S)
    errs2 = errs5.reshape(B, H, S)
    gidx = gidx5.reshape(B, H, S)
    c_count_hat = counts3.reshape(H, L)

    cb_flat = cb.reshape(H * L, D)
    cz = cb_flat[gidx.reshape(-1)]
    vecs_hat = cz.reshape(B, H, S, D)
    mask_rows = jnp.broadcast_to(loss_mask[:, None, :], (B, H, S)).reshape(-1)
    mv = vecs.reshape(-1, D) * mask_rows[:, None]
    c_sum_hat = jax.ops.segment_sum(mv, gidx.reshape(-1), num_segments=H * L)
    c_sum_hat = c_sum_hat.reshape(H, L, D)

    l_commit = jnp.mean(jnp.sum(loss_mask[:, None, :] * errs2, axis=1))
    c_sum_tgt = (1.0 - C_GAMMA) * w + C_GAMMA * c_sum_hat
    c_count_tgt = (1.0 - C_GAMMA) * c_count + C_GAMMA * c_count_hat
    l_codebook = (jnp.sum(jnp.square(w - c_sum_tgt))
                  + jnp.sum(jnp.square(c_count - c_count_tgt)))
    return (vecs_hat, z, l_commit, l_codebook)


# speedup vs baseline: 2.1977x; 1.3190x over previous
import functools

import jax
import jax.numpy as jnp
from jax import lax
from jax.experimental import pallas as pl
from jax.experimental.pallas import tpu as pltpu

B, H, S, D = 4, 8, 1024, 64
L = 1024
C_GAMMA = 0.99
EPS = 0.01
TS = 512
T = S // TS


def _a_body(vecs_ref, mask_ref, w_ref, cnt_ref,
            z_ref, errs_ref, cb_ref, gidx_ref, counts_ref):
    h = pl.program_id(0)
    b = pl.program_id(1)
    t = pl.program_id(2)

    @pl.when((b == 0) & (t == 0))
    def _init():
        cb_ref[0] = w_ref[0] / (cnt_ref[0, 0][:, None] + EPS)
        counts_ref[0, 0] = jnp.zeros((L,), jnp.float32)

    cb = cb_ref[0]
    cbn = jnp.sum(cb * cb, axis=1)
    v = vecs_ref[0, 0]
    vn = jnp.sum(v * v, axis=1, keepdims=True)
    scores = lax.dot_general(v, cb, (((1,), (1,)), ((), ())),
                             preferred_element_type=jnp.float32)
    d2 = (vn - 2.0 * scores) + cbn[None, :]
    m = jnp.min(d2, axis=1, keepdims=True)
    iota = lax.broadcasted_iota(jnp.int32, (TS, L), 1)
    z = jnp.min(jnp.where(d2 == m, iota, L), axis=1)
    z_ref[0, 0, 0, 0] = z
    gidx_ref[0, 0, 0, 0] = z + h * L
    errs_ref[0, 0, 0, 0] = m[:, 0]
    msk = mask_ref[0, 0, 0]
    onehot = jnp.where(z[:, None] == iota, msk[:, None], 0.0)
    counts_ref[0, 0] += jnp.sum(onehot, axis=0)


def _run_a(vecs, mask4, w, c_count):
    grid = (H, B, T)
    out_shapes = (
        jax.ShapeDtypeStruct((B, H, T, 1, TS), jnp.int32),
        jax.ShapeDtypeStruct((B, H, T, 1, TS), jnp.float32),
        jax.ShapeDtypeStruct((H, L, D), jnp.float32),
        jax.ShapeDtypeStruct((B, H, T, 1, TS), jnp.int32),
        jax.ShapeDtypeStruct((H, 1, L), jnp.float32),
    )
    in_specs = [
        pl.BlockSpec((1, 1, TS, D), lambda h, b, t: (b, h, t, 0)),
        pl.BlockSpec((1, 1, 1, TS), lambda h, b, t: (b, t, 0, 0)),
        pl.BlockSpec((1, L, D), lambda h, b, t: (h, 0, 0)),
        pl.BlockSpec((1, 1, L), lambda h, b, t: (h, 0, 0)),
    ]
    out_specs = (
        pl.BlockSpec((1, 1, 1, 1, TS), lambda h, b, t: (b, h, t, 0, 0)),
        pl.BlockSpec((1, 1, 1, 1, TS), lambda h, b, t: (b, h, t, 0, 0)),
        pl.BlockSpec((1, L, D), lambda h, b, t: (h, 0, 0)),
        pl.BlockSpec((1, 1, 1, 1, TS), lambda h, b, t: (b, h, t, 0, 0)),
        pl.BlockSpec((1, 1, L), lambda h, b, t: (h, 0, 0)),
    )
    return pl.pallas_call(
        _a_body,
        grid=grid,
        in_specs=in_specs,
        out_specs=out_specs,
        out_shape=out_shapes,
    )(vecs, mask4, w, c_count.reshape(H, 1, L))


NROW = B * H * S
RPW = NROW // 32
CHUNK = 128
NCH = RPW // CHUNK
HL = H * L


def _make_sc_kernel():
    from jax.experimental.pallas import tpu_sc as plsc

    mesh = plsc.VectorSubcoreMesh(core_axis_name="c", subcore_axis_name="s")

    @functools.partial(
        pl.kernel,
        mesh=mesh,
        out_type=[
            jax.ShapeDtypeStruct((NROW, D), jnp.float32),
            jax.ShapeDtypeStruct((2, HL, D), jnp.float32),
        ],
        scratch_types=[
            pltpu.VMEM((NCH, CHUNK), jnp.int32),
            pltpu.VMEM((CHUNK, D), jnp.float32),
            pltpu.VMEM((CHUNK, D), jnp.float32),
            pltpu.VMEM((CHUNK, D), jnp.float32),
            pltpu.VMEM_SHARED((HL, D), jnp.float32),
            pltpu.SemaphoreType.DMA,
        ],
        compiler_params=pltpu.CompilerParams(use_tc_tiling_on_sc=False),
    )
    def sc_b(gidx_hbm, cb_hbm, vecs_hbm, vh_hbm, parts_hbm,
             idx_v, v_buf, cz_buf, zbuf, acc, sem):
        c = lax.axis_index("c")
        s = lax.axis_index("s")
        wid = s * 2 + c
        base = wid * RPW

        def _zero_row(i, _):
            r = i // (D // 16)
            col = (i % (D // 16)) * 16
            zbuf[r, pl.ds(col, 16)] = jnp.zeros((16,), jnp.float32)
            return _
        lax.fori_loop(0, CHUNK * (D // 16), _zero_row, 0)
        for k in range(HL // (16 * CHUNK)):
            pltpu.sync_copy(zbuf, acc.at[pl.ds(s * (HL // 16) + k * CHUNK, CHUNK)])
        plsc.subcore_barrier()

        pltpu.sync_copy(gidx_hbm.at[pl.ds(wid * NCH, NCH)], idx_v)
        for j in range(NCH):
            row0 = base + j * CHUNK
            pltpu.sync_copy(vecs_hbm.at[pl.ds(row0, CHUNK)], v_buf)
            pltpu.async_copy(cb_hbm.at[idx_v.at[j]], cz_buf, sem).wait()
            pltpu.sync_copy(cz_buf, vh_hbm.at[pl.ds(row0, CHUNK)])
            pltpu.sync_copy(v_buf, acc.at[idx_v.at[j]], add=True)
        plsc.subcore_barrier()

        pltpu.sync_copy(acc.at[pl.ds(s * (HL // 16), HL // 16)],
                        parts_hbm.at[c, pl.ds(s * (HL // 16), HL // 16)])

    return sc_b


def _c_body(errs_ref, maskbh_ref, w2_ref, cc_ref, parts_ref, chat_ref,
            lcommit_ref, lcb_ref):
    lc = jnp.sum(maskbh_ref[...] * errs_ref[...]) * (1.0 / (B * S))
    lcommit_ref[...] = lc.reshape(1, 1)
    csum = parts_ref[0] + parts_ref[1]
    w2 = w2_ref[...]
    tgt = (1.0 - C_GAMMA) * w2 + C_GAMMA * csum
    s1 = jnp.sum(jnp.square(w2 - tgt))
    cc = cc_ref[...]
    ctgt = (1.0 - C_GAMMA) * cc + C_GAMMA * chat_ref[...]
    s2 = jnp.sum(jnp.square(cc - ctgt))
    lcb_ref[...] = (s1 + s2).reshape(1, 1)


def _run_c(errs2d, maskbh, w2, c_count, parts, c_count_hat):
    return pl.pallas_call(
        _c_body,
        out_shape=(jax.ShapeDtypeStruct((1, 1), jnp.float32),
                   jax.ShapeDtypeStruct((1, 1), jnp.float32)),
    )(errs2d, maskbh, w2, c_count, parts, c_count_hat)


def kernel(vecs, loss_mask, w, c_count):
    mask4 = loss_mask.reshape(B, T, 1, TS)
    z5, errs5, cb, gidx5, counts3 = _run_a(vecs, mask4, w, c_count)
    z = z5.reshape(B, H, S)
    c_count_hat = counts3.reshape(H, L)

    gidx2d = gidx5.reshape(NROW // CHUNK, CHUNK)
    vh_flat, parts = _make_sc_kernel()(gidx2d, cb.reshape(HL, D),
                                       vecs.reshape(NROW, D))
    vecs_hat = vh_flat.reshape(B, H, S, D)

    errs2d = errs5.reshape(B * H, S)
    maskbh = jnp.broadcast_to(loss_mask[:, None, :], (B, H, S)).reshape(B * H, S)
    lc, lcb = _run_c(errs2d, maskbh, w.reshape(HL, D), c_count,
                     parts, c_count_hat)
    return (vecs_hat, z, lc.reshape(()), lcb.reshape(()))


# speedup vs baseline: 2.4504x; 1.1150x over previous
import functools

import jax
import jax.numpy as jnp
from jax import lax
from jax.experimental import pallas as pl
from jax.experimental.pallas import tpu as pltpu

B, H, S, D = 4, 8, 1024, 64
L = 1024
C_GAMMA = 0.99
EPS = 0.01
TS = 1024
T = S // TS


def _a_body(vecs_ref, mask_ref, w_ref, cnt_ref,
            z_ref, errs_ref, cb_ref, gidx_ref, counts_ref, cbn_ref):
    h = pl.program_id(0)
    b = pl.program_id(1)

    @pl.when(b == 0)
    def _init():
        cb = w_ref[0] / (cnt_ref[0, 0][:, None] + EPS)
        cb_ref[0] = cb
        cbn_ref[0] = jnp.sum(cb * cb, axis=1)
        counts_ref[0, 0] = jnp.zeros((L,), jnp.float32)

    cb = cb_ref[0]
    cbn = cbn_ref[0]
    v = vecs_ref[0, 0]
    vn = jnp.sum(v * v, axis=1, keepdims=True)
    scores = lax.dot_general(v, cb, (((1,), (1,)), ((), ())),
                             preferred_element_type=jnp.float32)
    d2 = (vn - 2.0 * scores) + cbn[None, :]
    m = jnp.min(d2, axis=1)
    z = jnp.argmin(d2, axis=1).astype(jnp.int32)
    z_ref[0, 0, 0, 0] = z
    gidx_ref[0, 0, 0, 0] = z + h * L
    errs_ref[0, 0, 0, 0] = m
    msk = mask_ref[0, 0, 0]
    iota = lax.broadcasted_iota(jnp.int32, (TS, L), 1)
    onehot = jnp.where(z[:, None] == iota, msk[:, None], 0.0)
    counts_ref[0, 0] += jnp.sum(onehot, axis=0)


def _run_a(vecs, mask4, w, c_count):
    grid = (H, B * T)
    out_shapes = (
        jax.ShapeDtypeStruct((B, H, T, 1, TS), jnp.int32),
        jax.ShapeDtypeStruct((B, H, T, 1, TS), jnp.float32),
        jax.ShapeDtypeStruct((H, L, D), jnp.float32),
        jax.ShapeDtypeStruct((B, H, T, 1, TS), jnp.int32),
        jax.ShapeDtypeStruct((H, 1, L), jnp.float32),
    )
    in_specs = [
        pl.BlockSpec((1, 1, TS, D), lambda h, b: (b // T, h, b % T, 0)),
        pl.BlockSpec((1, 1, 1, TS), lambda h, b: (b // T, b % T, 0, 0)),
        pl.BlockSpec((1, L, D), lambda h, b: (h, 0, 0)),
        pl.BlockSpec((1, 1, L), lambda h, b: (h, 0, 0)),
    ]
    out_specs = (
        pl.BlockSpec((1, 1, 1, 1, TS), lambda h, b: (b // T, h, b % T, 0, 0)),
        pl.BlockSpec((1, 1, 1, 1, TS), lambda h, b: (b // T, h, b % T, 0, 0)),
        pl.BlockSpec((1, L, D), lambda h, b: (h, 0, 0)),
        pl.BlockSpec((1, 1, 1, 1, TS), lambda h, b: (b // T, h, b % T, 0, 0)),
        pl.BlockSpec((1, 1, L), lambda h, b: (h, 0, 0)),
    )
    return pl.pallas_call(
        _a_body,
        grid=grid,
        in_specs=in_specs,
        out_specs=out_specs,
        out_shape=out_shapes,
        scratch_shapes=[pltpu.VMEM((1, L), jnp.float32)],
    )(vecs, mask4, w, c_count.reshape(H, 1, L))


NROW = B * H * S
RPW = NROW // 32
CHUNK = 128
NCH = RPW // CHUNK
HL = H * L


def _make_sc_kernel():
    from jax.experimental.pallas import tpu_sc as plsc

    mesh = plsc.VectorSubcoreMesh(core_axis_name="c", subcore_axis_name="s")

    @functools.partial(
        pl.kernel,
        mesh=mesh,
        out_type=[
            jax.ShapeDtypeStruct((NROW, D), jnp.float32),
            jax.ShapeDtypeStruct((2, HL, D), jnp.float32),
        ],
        scratch_types=[
            pltpu.VMEM((NCH, CHUNK), jnp.int32),
            pltpu.VMEM((CHUNK, D), jnp.float32),
            pltpu.VMEM((CHUNK, D), jnp.float32),
            pltpu.VMEM((CHUNK, D), jnp.float32),
            pltpu.VMEM_SHARED((HL, D), jnp.float32),
            pltpu.SemaphoreType.DMA,
        ],
        compiler_params=pltpu.CompilerParams(use_tc_tiling_on_sc=False),
    )
    def sc_b(gidx_hbm, cb_hbm, vecs_hbm, vh_hbm, parts_hbm,
             idx_v, v_buf, cz_buf, zbuf, acc, sem):
        c = lax.axis_index("c")
        s = lax.axis_index("s")
        wid = s * 2 + c
        base = wid * RPW

        def _zero_row(i, _):
            r = i // (D // 16)
            col = (i % (D // 16)) * 16
            zbuf[r, pl.ds(col, 16)] = jnp.zeros((16,), jnp.float32)
            return _
        lax.fori_loop(0, CHUNK * (D // 16), _zero_row, 0)
        for k in range(HL // (16 * CHUNK)):
            pltpu.sync_copy(zbuf, acc.at[pl.ds(s * (HL // 16) + k * CHUNK, CHUNK)])
        plsc.subcore_barrier()

        pltpu.sync_copy(gidx_hbm.at[pl.ds(wid * NCH, NCH)], idx_v)
        for j in range(NCH):
            row0 = base + j * CHUNK
            pltpu.sync_copy(vecs_hbm.at[pl.ds(row0, CHUNK)], v_buf)
            pltpu.async_copy(cb_hbm.at[idx_v.at[j]], cz_buf, sem).wait()
            pltpu.sync_copy(cz_buf, vh_hbm.at[pl.ds(row0, CHUNK)])
            pltpu.sync_copy(v_buf, acc.at[idx_v.at[j]], add=True)
        plsc.subcore_barrier()

        pltpu.sync_copy(acc.at[pl.ds(s * (HL // 16), HL // 16)],
                        parts_hbm.at[c, pl.ds(s * (HL // 16), HL // 16)])

    return sc_b


def _c_body(errs_ref, maskbh_ref, w2_ref, cc_ref, parts_ref, chat_ref,
            lcommit_ref, lcb_ref):
    lc = jnp.sum(maskbh_ref[...] * errs_ref[...]) * (1.0 / (B * S))
    lcommit_ref[...] = lc.reshape(1, 1)
    csum = parts_ref[0] + parts_ref[1]
    w2 = w2_ref[...]
    tgt = (1.0 - C_GAMMA) * w2 + C_GAMMA * csum
    s1 = jnp.sum(jnp.square(w2 - tgt))
    cc = cc_ref[...]
    ctgt = (1.0 - C_GAMMA) * cc + C_GAMMA * chat_ref[...]
    s2 = jnp.sum(jnp.square(cc - ctgt))
    lcb_ref[...] = (s1 + s2).reshape(1, 1)


def _run_c(errs2d, maskbh, w2, c_count, parts, c_count_hat):
    return pl.pallas_call(
        _c_body,
        out_shape=(jax.ShapeDtypeStruct((1, 1), jnp.float32),
                   jax.ShapeDtypeStruct((1, 1), jnp.float32)),
    )(errs2d, maskbh, w2, c_count, parts, c_count_hat)


def kernel(vecs, loss_mask, w, c_count):
    mask4 = loss_mask.reshape(B, T, 1, TS)
    z5, errs5, cb, gidx5, counts3 = _run_a(vecs, mask4, w, c_count)
    z = z5.reshape(B, H, S)
    c_count_hat = counts3.reshape(H, L)

    gidx2d = gidx5.reshape(NROW // CHUNK, CHUNK)
    vh_flat, parts = _make_sc_kernel()(gidx2d, cb.reshape(HL, D),
                                       vecs.reshape(NROW, D))
    vecs_hat = vh_flat.reshape(B, H, S, D)

    errs2d = errs5.reshape(B * H, S)
    maskbh = jnp.broadcast_to(loss_mask[:, None, :], (B, H, S)).reshape(B * H, S)
    lc, lcb = _run_c(errs2d, maskbh, w.reshape(HL, D), c_count,
                     parts, c_count_hat)
    return (vecs_hat, z, lc.reshape(()), lcb.reshape(()))


# speedup vs baseline: 3.0179x; 1.2316x over previous
import functools

import jax
import jax.numpy as jnp
from jax import lax
from jax.experimental import pallas as pl
from jax.experimental.pallas import tpu as pltpu

B, H, S, D = 4, 8, 1024, 64
L = 1024
C_GAMMA = 0.99
EPS = 0.01
TS = 1024
T = S // TS


def _a_body(vecs_ref, mask_ref, w_ref, cnt_ref,
            z_ref, errs_ref, cb_ref, gidx_ref, counts_ref, cbn_ref, eye_ref):
    h = pl.program_id(0)
    b = pl.program_id(1)

    @pl.when((h == 0) & (b == 0))
    def _eye():
        ri = lax.broadcasted_iota(jnp.int32, (TS, TS), 0)
        ci = lax.broadcasted_iota(jnp.int32, (TS, TS), 1)
        eye_ref[...] = jnp.where(ri == ci, 1.0, 0.0).astype(jnp.float32)

    @pl.when(b == 0)
    def _init():
        cb = w_ref[0] / (cnt_ref[0, 0][:, None] + EPS)
        cb_ref[0] = cb
        cbn_ref[...] = jnp.sum(cb * cb, axis=1, keepdims=True)
        counts_ref[0] = jnp.zeros((L, 1), jnp.float32)

    cb = cb_ref[0]
    cbn = cbn_ref[...]
    v = vecs_ref[0, 0]
    vn = jnp.sum(v * v, axis=1, keepdims=True)
    vn_lane = lax.dot_general(vn, eye_ref[...], (((0,), (0,)), ((), ())),
                              preferred_element_type=jnp.float32)
    scores_t = lax.dot_general(cb, v, (((1,), (1,)), ((), ())),
                               preferred_element_type=jnp.float32)
    d2 = (vn_lane - 2.0 * scores_t) + cbn
    m = jnp.min(d2, axis=0)
    iota = lax.broadcasted_iota(jnp.int32, (L, TS), 0)
    z = jnp.argmin(d2, axis=0).astype(jnp.int32)
    z_ref[0, 0, 0, 0] = z
    gidx_ref[0, 0, 0, 0] = z + h * L
    errs_ref[0, 0, 0, 0] = m
    msk = mask_ref[0, 0, 0]
    onehot_t = jnp.where(z[None, :] == iota, msk[None, :], 0.0)
    ones_row = jnp.ones((1, TS), jnp.float32)
    counts_ref[0] += lax.dot_general(
        onehot_t, ones_row, (((1,), (1,)), ((), ())),
        preferred_element_type=jnp.float32)


def _run_a(vecs, mask4, w, c_count):
    grid = (H, B * T)
    out_shapes = (
        jax.ShapeDtypeStruct((B, H, T, 1, TS), jnp.int32),
        jax.ShapeDtypeStruct((B, H, T, 1, TS), jnp.float32),
        jax.ShapeDtypeStruct((H, L, D), jnp.float32),
        jax.ShapeDtypeStruct((B, H, T, 1, TS), jnp.int32),
        jax.ShapeDtypeStruct((H, L, 1), jnp.float32),
    )
    in_specs = [
        pl.BlockSpec((1, 1, TS, D), lambda h, b: (b // T, h, b % T, 0)),
        pl.BlockSpec((1, 1, 1, TS), lambda h, b: (b // T, b % T, 0, 0)),
        pl.BlockSpec((1, L, D), lambda h, b: (h, 0, 0)),
        pl.BlockSpec((1, 1, L), lambda h, b: (h, 0, 0)),
    ]
    out_specs = (
        pl.BlockSpec((1, 1, 1, 1, TS), lambda h, b: (b // T, h, b % T, 0, 0)),
        pl.BlockSpec((1, 1, 1, 1, TS), lambda h, b: (b // T, h, b % T, 0, 0)),
        pl.BlockSpec((1, L, D), lambda h, b: (h, 0, 0)),
        pl.BlockSpec((1, 1, 1, 1, TS), lambda h, b: (b // T, h, b % T, 0, 0)),
        pl.BlockSpec((1, L, 1), lambda h, b: (h, 0, 0)),
    )
    return pl.pallas_call(
        _a_body,
        grid=grid,
        in_specs=in_specs,
        out_specs=out_specs,
        out_shape=out_shapes,
        scratch_shapes=[pltpu.VMEM((L, 1), jnp.float32),
                        pltpu.VMEM((TS, TS), jnp.float32)],
    )(vecs, mask4, w, c_count.reshape(H, 1, L))


NROW = B * H * S
RPW = NROW // 32
CHUNK = 128
NCH = RPW // CHUNK
HL = H * L


def _make_sc_kernel():
    from jax.experimental.pallas import tpu_sc as plsc

    mesh = plsc.VectorSubcoreMesh(core_axis_name="c", subcore_axis_name="s")

    @functools.partial(
        pl.kernel,
        mesh=mesh,
        out_type=[
            jax.ShapeDtypeStruct((NROW, D), jnp.float32),
            jax.ShapeDtypeStruct((2, HL, D), jnp.float32),
        ],
        scratch_types=[
            pltpu.VMEM((NCH, CHUNK), jnp.int32),
            pltpu.VMEM((CHUNK, D), jnp.float32),
            pltpu.VMEM((CHUNK, D), jnp.float32),
            pltpu.VMEM((CHUNK, D), jnp.float32),
            pltpu.VMEM_SHARED((HL, D), jnp.float32),
            pltpu.SemaphoreType.DMA,
        ],
        compiler_params=pltpu.CompilerParams(use_tc_tiling_on_sc=False),
    )
    def sc_b(gidx_hbm, cb_hbm, vecs_hbm, vh_hbm, parts_hbm,
             idx_v, v_buf, cz_buf, zbuf, acc, sem):
        c = lax.axis_index("c")
        s = lax.axis_index("s")
        wid = s * 2 + c
        base = wid * RPW

        def _zero_row(i, _):
            r = i // (D // 16)
            col = (i % (D // 16)) * 16
            zbuf[r, pl.ds(col, 16)] = jnp.zeros((16,), jnp.float32)
            return _
        lax.fori_loop(0, CHUNK * (D // 16), _zero_row, 0)
        for k in range(HL // (16 * CHUNK)):
            pltpu.sync_copy(zbuf, acc.at[pl.ds(s * (HL // 16) + k * CHUNK, CHUNK)])
        plsc.subcore_barrier()

        pltpu.sync_copy(gidx_hbm.at[pl.ds(wid * NCH, NCH)], idx_v)
        for j in range(NCH):
            row0 = base + j * CHUNK
            pltpu.sync_copy(vecs_hbm.at[pl.ds(row0, CHUNK)], v_buf)
            pltpu.async_copy(cb_hbm.at[idx_v.at[j]], cz_buf, sem).wait()
            pltpu.sync_copy(cz_buf, vh_hbm.at[pl.ds(row0, CHUNK)])
            pltpu.sync_copy(v_buf, acc.at[idx_v.at[j]], add=True)
        plsc.subcore_barrier()

        pltpu.sync_copy(acc.at[pl.ds(s * (HL // 16), HL // 16)],
                        parts_hbm.at[c, pl.ds(s * (HL // 16), HL // 16)])

    return sc_b


def _c_body(errs_ref, maskbh_ref, w2_ref, cc_ref, parts_ref, chat_ref,
            lcommit_ref, lcb_ref):
    lc = jnp.sum(maskbh_ref[...] * errs_ref[...]) * (1.0 / (B * S))
    lcommit_ref[...] = lc.reshape(1, 1)
    csum = parts_ref[0] + parts_ref[1]
    w2 = w2_ref[...]
    tgt = (1.0 - C_GAMMA) * w2 + C_GAMMA * csum
    s1 = jnp.sum(jnp.square(w2 - tgt))
    cc = cc_ref[...]
    ctgt = (1.0 - C_GAMMA) * cc + C_GAMMA * chat_ref[...]
    s2 = jnp.sum(jnp.square(cc - ctgt))
    lcb_ref[...] = (s1 + s2).reshape(1, 1)


def _run_c(errs2d, maskbh, w2, c_count, parts, c_count_hat):
    return pl.pallas_call(
        _c_body,
        out_shape=(jax.ShapeDtypeStruct((1, 1), jnp.float32),
                   jax.ShapeDtypeStruct((1, 1), jnp.float32)),
    )(errs2d, maskbh, w2, c_count, parts, c_count_hat)


def kernel(vecs, loss_mask, w, c_count):
    mask4 = loss_mask.reshape(B, T, 1, TS)
    z5, errs5, cb, gidx5, counts3 = _run_a(vecs, mask4, w, c_count)
    z = z5.reshape(B, H, S)
    c_count_hat = counts3.reshape(H, L)

    gidx2d = gidx5.reshape(NROW // CHUNK, CHUNK)
    vh_flat, parts = _make_sc_kernel()(gidx2d, cb.reshape(HL, D),
                                       vecs.reshape(NROW, D))
    vecs_hat = vh_flat.reshape(B, H, S, D)

    errs2d = errs5.reshape(B * H, S)
    maskbh = jnp.broadcast_to(loss_mask[:, None, :], (B, H, S)).reshape(B * H, S)
    lc, lcb = _run_c(errs2d, maskbh, w.reshape(HL, D), c_count,
                     parts, c_count_hat)
    return (vecs_hat, z, lc.reshape(()), lcb.reshape(()))


# speedup vs baseline: 3.1954x; 1.0588x over previous
import functools

import jax
import jax.numpy as jnp
from jax import lax
from jax.experimental import pallas as pl
from jax.experimental.pallas import tpu as pltpu

B, H, S, D = 4, 8, 1024, 64
L = 1024
C_GAMMA = 0.99
EPS = 0.01
TS = 1024
T = S // TS


def _a_body(vecs_ref, mask_ref, w_ref, cnt_ref,
            z_ref, errs_ref, cb_ref, gidx_ref, counts_ref, cbn_ref, eye_ref):
    h = pl.program_id(0)
    b = pl.program_id(1)

    @pl.when((h == 0) & (b == 0))
    def _eye():
        ri = lax.broadcasted_iota(jnp.int32, (TS, TS), 0)
        ci = lax.broadcasted_iota(jnp.int32, (TS, TS), 1)
        eye_ref[...] = jnp.where(ri == ci, 1.0, 0.0).astype(jnp.float32)

    @pl.when(b == 0)
    def _init():
        cb = w_ref[0] / (cnt_ref[0, 0][:, None] + EPS)
        cb_ref[0] = cb
        cbn_ref[...] = jnp.sum(cb * cb, axis=1, keepdims=True)
        counts_ref[0] = jnp.zeros((L, 1), jnp.float32)

    cb = cb_ref[0]
    cbn = cbn_ref[...]
    v = vecs_ref[0, 0]
    vn = jnp.sum(v * v, axis=1, keepdims=True)
    vn_lane = lax.dot_general(vn, eye_ref[...], (((0,), (0,)), ((), ())),
                              preferred_element_type=jnp.float32)
    scores_t = lax.dot_general(cb, v, (((1,), (1,)), ((), ())),
                               preferred_element_type=jnp.float32)
    d2 = (vn_lane - 2.0 * scores_t) + cbn
    m = jnp.min(d2, axis=0)
    iota = lax.broadcasted_iota(jnp.int32, (L, TS), 0)
    z = jnp.argmin(d2, axis=0).astype(jnp.int32)
    z_ref[0, 0, 0, 0] = z
    gidx_ref[0, 0, 0, 0] = z + h * L
    errs_ref[0, 0, 0, 0] = m
    msk = mask_ref[0, 0, 0]
    onehot_t = jnp.where(z[None, :] == iota, msk[None, :], 0.0)
    ones_row = jnp.ones((1, TS), jnp.float32)
    counts_ref[0] += lax.dot_general(
        onehot_t, ones_row, (((1,), (1,)), ((), ())),
        preferred_element_type=jnp.float32)


def _run_a(vecs, mask4, w, c_count):
    grid = (H, B * T)
    out_shapes = (
        jax.ShapeDtypeStruct((B, H, T, 1, TS), jnp.int32),
        jax.ShapeDtypeStruct((B, H, T, 1, TS), jnp.float32),
        jax.ShapeDtypeStruct((H, L, D), jnp.float32),
        jax.ShapeDtypeStruct((B, H, T, 1, TS), jnp.int32),
        jax.ShapeDtypeStruct((H, L, 1), jnp.float32),
    )
    in_specs = [
        pl.BlockSpec((1, 1, TS, D), lambda h, b: (b // T, h, b % T, 0)),
        pl.BlockSpec((1, 1, 1, TS), lambda h, b: (b // T, b % T, 0, 0)),
        pl.BlockSpec((1, L, D), lambda h, b: (h, 0, 0)),
        pl.BlockSpec((1, 1, L), lambda h, b: (h, 0, 0)),
    ]
    out_specs = (
        pl.BlockSpec((1, 1, 1, 1, TS), lambda h, b: (b // T, h, b % T, 0, 0)),
        pl.BlockSpec((1, 1, 1, 1, TS), lambda h, b: (b // T, h, b % T, 0, 0)),
        pl.BlockSpec((1, L, D), lambda h, b: (h, 0, 0)),
        pl.BlockSpec((1, 1, 1, 1, TS), lambda h, b: (b // T, h, b % T, 0, 0)),
        pl.BlockSpec((1, L, 1), lambda h, b: (h, 0, 0)),
    )
    return pl.pallas_call(
        _a_body,
        grid=grid,
        in_specs=in_specs,
        out_specs=out_specs,
        out_shape=out_shapes,
        scratch_shapes=[pltpu.VMEM((L, 1), jnp.float32),
                        pltpu.VMEM((TS, TS), jnp.float32)],
    )(vecs, mask4, w, c_count.reshape(H, 1, L))


NROW = B * H * S
RPW = NROW // 32
CHUNK = 128
NCH = RPW // CHUNK
HL = H * L


def _make_sc_kernel():
    from jax.experimental.pallas import tpu_sc as plsc

    mesh = plsc.VectorSubcoreMesh(core_axis_name="c", subcore_axis_name="s")

    @functools.partial(
        pl.kernel,
        mesh=mesh,
        out_type=[
            jax.ShapeDtypeStruct((NROW, D), jnp.float32),
            jax.ShapeDtypeStruct((2, HL, D), jnp.float32),
        ],
        scratch_types=[
            pltpu.VMEM((NCH, CHUNK), jnp.int32),
            pltpu.VMEM((4, CHUNK, D), jnp.float32),
            pltpu.VMEM((4, CHUNK, D), jnp.float32),
            pltpu.VMEM((CHUNK, D), jnp.float32),
            pltpu.VMEM_SHARED((HL, D), jnp.float32),
            pltpu.SemaphoreType.DMA,
            pltpu.SemaphoreType.DMA,
            pltpu.SemaphoreType.DMA,
            pltpu.SemaphoreType.DMA,
        ],
        compiler_params=pltpu.CompilerParams(use_tc_tiling_on_sc=False),
    )
    def sc_b(gidx_hbm, cb_hbm, vecs_hbm, vh_hbm, parts_hbm,
             idx_v, v_buf, cz_buf, zbuf, acc, sem_v, sem_g, sem_s, sem_a):
        c = lax.axis_index("c")
        s = lax.axis_index("s")
        wid = s * 2 + c
        base = wid * RPW

        def _zero_row(i, _):
            r = i // (D // 16)
            col = (i % (D // 16)) * 16
            zbuf[r, pl.ds(col, 16)] = jnp.zeros((16,), jnp.float32)
            return _
        lax.fori_loop(0, CHUNK * (D // 16), _zero_row, 0)
        pltpu.sync_copy(gidx_hbm.at[pl.ds(wid * NCH, NCH)], idx_v)
        for k in range(HL // (16 * CHUNK)):
            pltpu.sync_copy(zbuf, acc.at[pl.ds(s * (HL // 16) + k * CHUNK, CHUNK)])
        plsc.subcore_barrier()

        prev = []
        for batch in range(NCH // 4):
            for cpy in prev:
                cpy.wait()
            prev = []
            vc, gc = [], []
            for j4 in range(4):
                j = batch * 4 + j4
                row0 = base + j * CHUNK
                vc.append(pltpu.async_copy(vecs_hbm.at[pl.ds(row0, CHUNK)],
                                           v_buf.at[j4], sem_v))
                gc.append(pltpu.async_copy(cb_hbm.at[idx_v.at[j]],
                                           cz_buf.at[j4], sem_g))
            for j4 in range(4):
                j = batch * 4 + j4
                row0 = base + j * CHUNK
                gc[j4].wait()
                prev.append(pltpu.async_copy(cz_buf.at[j4],
                                             vh_hbm.at[pl.ds(row0, CHUNK)],
                                             sem_s))
                vc[j4].wait()
                prev.append(pltpu.async_copy(v_buf.at[j4], acc.at[idx_v.at[j]],
                                             sem_a, add=True))
        for cpy in prev:
            cpy.wait()
        plsc.subcore_barrier()

        pltpu.sync_copy(acc.at[pl.ds(s * (HL // 16), HL // 16)],
                        parts_hbm.at[c, pl.ds(s * (HL // 16), HL // 16)])

    return sc_b


def _c_body(errs_ref, maskbh_ref, w2_ref, cc_ref, parts_ref, chat_ref,
            lcommit_ref, lcb_ref):
    lc = jnp.sum(maskbh_ref[...] * errs_ref[...]) * (1.0 / (B * S))
    lcommit_ref[...] = lc.reshape(1, 1)
    csum = parts_ref[0] + parts_ref[1]
    w2 = w2_ref[...]
    tgt = (1.0 - C_GAMMA) * w2 + C_GAMMA * csum
    s1 = jnp.sum(jnp.square(w2 - tgt))
    cc = cc_ref[...]
    ctgt = (1.0 - C_GAMMA) * cc + C_GAMMA * chat_ref[...]
    s2 = jnp.sum(jnp.square(cc - ctgt))
    lcb_ref[...] = (s1 + s2).reshape(1, 1)


def _run_c(errs2d, maskbh, w2, c_count, parts, c_count_hat):
    return pl.pallas_call(
        _c_body,
        out_shape=(jax.ShapeDtypeStruct((1, 1), jnp.float32),
                   jax.ShapeDtypeStruct((1, 1), jnp.float32)),
    )(errs2d, maskbh, w2, c_count, parts, c_count_hat)


def kernel(vecs, loss_mask, w, c_count):
    mask4 = loss_mask.reshape(B, T, 1, TS)
    z5, errs5, cb, gidx5, counts3 = _run_a(vecs, mask4, w, c_count)
    z = z5.reshape(B, H, S)
    c_count_hat = counts3.reshape(H, L)

    gidx2d = gidx5.reshape(NROW // CHUNK, CHUNK)
    vh_flat, parts = _make_sc_kernel()(gidx2d, cb.reshape(HL, D),
                                       vecs.reshape(NROW, D))
    vecs_hat = vh_flat.reshape(B, H, S, D)

    errs2d = errs5.reshape(B * H, S)
    maskbh = jnp.broadcast_to(loss_mask[:, None, :], (B, H, S)).reshape(B * H, S)
    lc, lcb = _run_c(errs2d, maskbh, w.reshape(HL, D), c_count,
                     parts, c_count_hat)
    return (vecs_hat, z, lc.reshape(()), lcb.reshape(()))
